# baseline (speedup 1.0000x reference)
import functools

import jax
import jax.numpy as jnp
from jax import lax
from jax.experimental import pallas as pl
from jax.experimental.pallas import tpu as pltpu
from jax.experimental.pallas import tpu_sc as plsc

_B = 16
_T = 66
_L = 3
_F = 1024
_D = 64
_K = 16
_TT = _T - _L + 1
_N = _B * _TT
_MOMENTUM = 0.99
_EPS = 1e-8
_BETA = 1.0
_SCALE = 1.0

_NC = 2
_NS = 16
_NW = _NC * _NS
_RPW = _N // _NW
_R = 4
_NCHUNK = _N // 16


def _d2_kernel(feat_ref, proj_ref, d2_ref):
    f = feat_ref[...]
    w = (f[:, 0:_TT, :] + f[:, 1:_TT + 1, :] + f[:, 2:_TT + 2, :]) * (1.0 / _L)
    w2 = w.reshape(_N, _F)
    sf = jnp.dot(w2, proj_ref[...], preferred_element_type=jnp.float32)
    g = jax.lax.dot_general(sf, sf, (((1,), (1,)), ((), ())),
                            preferred_element_type=jnp.float32)
    n2 = jnp.sum(sf * sf, axis=1, keepdims=True)
    d2_ref[...] = jnp.maximum(n2 + n2.reshape(1, _N) - 2.0 * g, 0.0)


def _sc_sort(x):
    return plsc.sort_key_val(x, x)[0]


def _sc_topk_body(d2_hbm, out_hbm, rows_v, out_v):
    wid = lax.axis_index("s") * _NC + lax.axis_index("c")
    base = wid * _RPW
    pltpu.sync_copy(d2_hbm.at[pl.ds(base, _RPW)], rows_v)
    for rb in range(_RPW // _R):
        bests = tuple(
            _sc_sort(rows_v[rb * _R + r, pl.ds(0, 16)]) for r in range(_R)
        )

        def body(j, bs, _rb=rb):
            nb = []
            for r in range(_R):
                c = _sc_sort(rows_v[_rb * _R + r, pl.ds(j * 16, 16)])
                nb.append(_sc_sort(jnp.minimum(bs[r], lax.rev(c, (0,)))))
            return tuple(nb)

        bests = lax.fori_loop(1, _NCHUNK, body, bests)
        for r in range(_R):
            out_v[rb * _R + r, :] = bests[r]
    pltpu.sync_copy(out_v, out_hbm.at[pl.ds(base, _RPW)])


_sc_topk = functools.partial(
    pl.kernel,
    out_type=jax.ShapeDtypeStruct((_N, _K), jnp.float32),
    mesh=plsc.VectorSubcoreMesh(core_axis_name="c", subcore_axis_name="s"),
    scratch_types=[
        pltpu.VMEM((_RPW, _N), jnp.float32),
        pltpu.VMEM((_RPW, _K), jnp.float32),
    ],
    compiler_params=pltpu.CompilerParams(needs_layout_passes=False),
)(_sc_topk_body)


def _finish_kernel(top_ref, rew_ref, out_ref):
    dist = jnp.sqrt(top_ref[...])
    int_rew = jnp.sum(dist, axis=1, keepdims=True) * (1.0 / _K)
    mag = _MOMENTUM + (1.0 - _MOMENTUM) * jnp.mean(jnp.abs(int_rew))
    out_ref[...] = rew_ref[...] + _BETA * _SCALE * int_rew / (mag + _EPS)


@jax.jit
def kernel(reward, feat, proj, queue):
    del queue
    rew2 = reward[:, :_TT].reshape(_N, 1)
    d2 = pl.pallas_call(
        _d2_kernel,
        out_shape=jax.ShapeDtypeStruct((_N, _N), jnp.float32),
    )(feat, proj)
    top = _sc_topk(d2)
    out = pl.pallas_call(
        _finish_kernel,
        out_shape=jax.ShapeDtypeStruct((_N, 1), jnp.float32),
    )(top, rew2)
    return out.reshape(_B, _TT, 1)
